# Initial kernel scaffold; baseline (speedup 1.0000x reference)
import jax
import jax.numpy as jnp
from jax.experimental import pallas as pl


def kernel(x, edge_index, W1_l, b1, W1_r, W2_l, b2, W2_r):
    raise NotImplementedError("write your pallas kernel here")


# speedup vs baseline: 12.4926x; 12.4926x over previous
import functools

import jax
import jax.numpy as jnp
from jax import lax
from jax.experimental import pallas as pl
from jax.experimental.pallas import tpu as pltpu
from jax.experimental.pallas import tpu_sc as plsc

_N = 10000
_E = 320000
_DF = 128
_DH = 16
_NCLS = 3

_NCORES = 2
_NSUB = 16
_NW = _NCORES * _NSUB
_CHUNK = 128
_EPT = _E // _NW
_NCHUNK = -(-_EPT // _CHUNK)
_EPT_PAD = _NCHUNK * _CHUNK
_RPT = 640
_NPAD = _RPT * _NSUB

_BM = 2000


def _proj1_body(x_ref, wl_ref, wr_ref, b_ref, y_ref, r_ref):
    xb = x_ref[...]
    y_ref[...] = jnp.dot(xb, wl_ref[...], preferred_element_type=jnp.float32)
    r_ref[...] = (jnp.dot(xb, wr_ref[...], preferred_element_type=jnp.float32)
                  + b_ref[...])


def _mid_body(s_ref, d_ref, r_ref, wl_ref, wr_ref, b_ref, y_ref, r2_ref):
    s = s_ref[0] + s_ref[1]
    d = jnp.maximum(d_ref[0] + d_ref[1], 1.0)
    h = jnp.maximum(s / d + r_ref[...], 0.0)
    y_ref[...] = jnp.dot(h, wl_ref[...], preferred_element_type=jnp.float32)
    r2_ref[...] = (jnp.dot(h, wr_ref[...], preferred_element_type=jnp.float32)
                   + b_ref[...])


def _out_body(s_ref, d_ref, r_ref, o_ref):
    s = s_ref[0] + s_ref[1]
    d = jnp.maximum(d_ref[0] + d_ref[1], 1.0)
    z = jnp.maximum(s / d + r_ref[...], 0.0)
    col = lax.broadcasted_iota(jnp.int32, z.shape, 1)
    mask = col < _NCLS
    zm = jnp.where(mask, z, -1e30)
    m = jnp.max(zm, axis=1, keepdims=True)
    e = jnp.where(mask, jnp.exp(z - m), 0.0)
    se = jnp.sum(e, axis=1, keepdims=True)
    o_ref[...] = z - m - jnp.log(se)


_GRID = _N // _BM

_proj1 = pl.pallas_call(
    _proj1_body,
    grid=(_GRID,),
    in_specs=[
        pl.BlockSpec((_BM, _DF), lambda i: (i, 0)),
        pl.BlockSpec((_DF, _DH), lambda i: (0, 0)),
        pl.BlockSpec((_DF, _DH), lambda i: (0, 0)),
        pl.BlockSpec((1, _DH), lambda i: (0, 0)),
    ],
    out_specs=[
        pl.BlockSpec((_BM, _DH), lambda i: (i, 0)),
        pl.BlockSpec((_BM, _DH), lambda i: (i, 0)),
    ],
    out_shape=[
        jax.ShapeDtypeStruct((_N, _DH), jnp.float32),
        jax.ShapeDtypeStruct((_N, _DH), jnp.float32),
    ],
)

_mid = pl.pallas_call(
    _mid_body,
    grid=(_GRID,),
    in_specs=[
        pl.BlockSpec((_NCORES, _BM, _DH), lambda i: (0, i, 0)),
        pl.BlockSpec((_NCORES, _BM, _DH), lambda i: (0, i, 0)),
        pl.BlockSpec((_BM, _DH), lambda i: (i, 0)),
        pl.BlockSpec((_DH, _DH), lambda i: (0, 0)),
        pl.BlockSpec((_DH, _DH), lambda i: (0, 0)),
        pl.BlockSpec((1, _DH), lambda i: (0, 0)),
    ],
    out_specs=[
        pl.BlockSpec((_BM, _DH), lambda i: (i, 0)),
        pl.BlockSpec((_BM, _DH), lambda i: (i, 0)),
    ],
    out_shape=[
        jax.ShapeDtypeStruct((_N, _DH), jnp.float32),
        jax.ShapeDtypeStruct((_N, _DH), jnp.float32),
    ],
)

_outk = pl.pallas_call(
    _out_body,
    grid=(_GRID,),
    in_specs=[
        pl.BlockSpec((_NCORES, _BM, _DH), lambda i: (0, i, 0)),
        pl.BlockSpec((_NCORES, _BM, _DH), lambda i: (0, i, 0)),
        pl.BlockSpec((_BM, _DH), lambda i: (i, 0)),
    ],
    out_specs=pl.BlockSpec((_BM, _DH), lambda i: (i, 0)),
    out_shape=jax.ShapeDtypeStruct((_N, _DH), jnp.float32),
)


_mesh = plsc.VectorSubcoreMesh(core_axis_name="c", subcore_axis_name="s")


def _make_seg_kernel(with_deg: bool):
    acc_t = jax.ShapeDtypeStruct((_NCORES, _NPAD, _DH), jnp.float32)
    out_type = [acc_t, acc_t] if with_deg else acc_t
    scratch = [
        pltpu.VMEM((_NCHUNK, _CHUNK), jnp.int32),
        pltpu.VMEM((_NCHUNK, _CHUNK), jnp.int32),
        pltpu.VMEM((_CHUNK, _DH), jnp.float32),
        pltpu.VMEM_SHARED((_NPAD, _DH), jnp.float32),
        pltpu.SemaphoreType.DMA,
    ]
    if with_deg:
        scratch.insert(3, pltpu.VMEM((_CHUNK, _DH), jnp.float32))
        scratch.insert(5, pltpu.VMEM_SHARED((_NPAD, _DH), jnp.float32))

    def body(vals_hbm, src_hbm, dst_hbm, zeros_hbm, *rest):
        if with_deg:
            (ones_hbm, out_hbm, deg_hbm,
             src_v, dst_v, rows_v, ones_v, acc_sh, deg_sh, sem) = rest
        else:
            (out_hbm, src_v, dst_v, rows_v, acc_sh, sem) = rest
        cid = lax.axis_index("c")
        sid = lax.axis_index("s")
        wid = cid * _NSUB + sid
        pltpu.sync_copy(src_hbm.at[wid], src_v)
        pltpu.sync_copy(dst_hbm.at[wid], dst_v)
        if with_deg:
            pltpu.sync_copy(ones_hbm, ones_v)
        r0 = sid * _RPT
        pltpu.sync_copy(zeros_hbm.at[pl.ds(r0, _RPT)],
                        acc_sh.at[pl.ds(r0, _RPT)])
        if with_deg:
            pltpu.sync_copy(zeros_hbm.at[pl.ds(r0, _RPT)],
                            deg_sh.at[pl.ds(r0, _RPT)])
        plsc.subcore_barrier()

        def chunk(j, carry):
            pltpu.async_copy(vals_hbm.at[src_v.at[j]], rows_v, sem).wait()
            pltpu.sync_copy(rows_v, acc_sh.at[dst_v.at[j]], add=True)
            if with_deg:
                pltpu.sync_copy(ones_v, deg_sh.at[dst_v.at[j]], add=True)
            return carry

        lax.fori_loop(0, _NCHUNK, chunk, 0)
        plsc.subcore_barrier()
        pltpu.sync_copy(acc_sh.at[pl.ds(r0, _RPT)],
                        out_hbm.at[cid, pl.ds(r0, _RPT)])
        if with_deg:
            pltpu.sync_copy(deg_sh.at[pl.ds(r0, _RPT)],
                            deg_hbm.at[cid, pl.ds(r0, _RPT)])

    return functools.partial(
        pl.kernel, mesh=_mesh, out_type=out_type, scratch_types=scratch,
        compiler_params=pltpu.CompilerParams(use_tc_tiling_on_sc=False),
    )(body)


_seg_deg = _make_seg_kernel(with_deg=True)
_seg = _make_seg_kernel(with_deg=False)


def kernel(x, edge_index, W1_l, b1, W1_r, W2_l, b2, W2_r):
    f32 = jnp.float32
    src = edge_index[0].reshape(_NW, _EPT)
    dst = edge_index[1].reshape(_NW, _EPT)
    pad = _EPT_PAD - _EPT
    src = jnp.pad(src, ((0, 0), (0, pad))).reshape(_NW, _NCHUNK, _CHUNK)
    dst = jnp.pad(dst, ((0, 0), (0, pad)), constant_values=_N)
    dst = dst.reshape(_NW, _NCHUNK, _CHUNK)

    zeros = jnp.zeros((_NPAD, _DH), f32)
    ones = jnp.ones((_CHUNK, _DH), f32)

    y1, r1 = _proj1(x, W1_l, W1_r, b1.reshape(1, _DH))
    s1, deg = _seg_deg(y1, src, dst, zeros, ones)

    w2l = jnp.pad(W2_l, ((0, 0), (0, _DH - _NCLS)))
    w2r = jnp.pad(W2_r, ((0, 0), (0, _DH - _NCLS)))
    b2p = jnp.pad(b2, (0, _DH - _NCLS)).reshape(1, _DH)
    y2, r2 = _mid(s1, deg, r1, w2l, w2r, b2p)

    s2 = _seg(y2, src, dst, zeros)
    out = _outk(s2, deg, r2)
    return out[:, :_NCLS]


# speedup vs baseline: 20.3459x; 1.6286x over previous
import functools

import jax
import jax.numpy as jnp
from jax import lax
from jax.experimental import pallas as pl
from jax.experimental.pallas import tpu as pltpu
from jax.experimental.pallas import tpu_sc as plsc

_N = 10000
_E = 320000
_DF = 128
_DH = 16
_NCLS = 3

_NCORES = 2
_NSUB = 16
_NW = _NCORES * _NSUB
_CHUNK = 128
_EPT = _E // _NW
_NCHUNK = -(-_EPT // _CHUNK)
_EPT_PAD = _NCHUNK * _CHUNK
_RPT = 640
_NPAD = _RPT * _NSUB

_BM = 2000


def _proj1_body(x_ref, wl_ref, wr_ref, b_ref, y_ref, r_ref):
    xb = x_ref[...]
    y_ref[...] = jnp.dot(xb, wl_ref[...], preferred_element_type=jnp.float32)
    r_ref[...] = (jnp.dot(xb, wr_ref[...], preferred_element_type=jnp.float32)
                  + b_ref[...])


def _mid_body(s_ref, d_ref, r_ref, wl_ref, wr_ref, b_ref, y_ref, r2_ref):
    s = s_ref[0] + s_ref[1]
    d = jnp.maximum(d_ref[0] + d_ref[1], 1.0)
    h = jnp.maximum(s / d + r_ref[...], 0.0)
    y_ref[...] = jnp.dot(h, wl_ref[...], preferred_element_type=jnp.float32)
    r2_ref[...] = (jnp.dot(h, wr_ref[...], preferred_element_type=jnp.float32)
                   + b_ref[...])


def _out_body(s_ref, d_ref, r_ref, o_ref):
    s = s_ref[0] + s_ref[1]
    d = jnp.maximum(d_ref[0] + d_ref[1], 1.0)
    z = jnp.maximum(s / d + r_ref[...], 0.0)
    col = lax.broadcasted_iota(jnp.int32, z.shape, 1)
    mask = col < _NCLS
    zm = jnp.where(mask, z, -1e30)
    m = jnp.max(zm, axis=1, keepdims=True)
    e = jnp.where(mask, jnp.exp(z - m), 0.0)
    se = jnp.sum(e, axis=1, keepdims=True)
    o_ref[...] = z - m - jnp.log(se)


_GRID = _N // _BM

_proj1 = pl.pallas_call(
    _proj1_body,
    grid=(_GRID,),
    in_specs=[
        pl.BlockSpec((_BM, _DF), lambda i: (i, 0)),
        pl.BlockSpec((_DF, _DH), lambda i: (0, 0)),
        pl.BlockSpec((_DF, _DH), lambda i: (0, 0)),
        pl.BlockSpec((1, _DH), lambda i: (0, 0)),
    ],
    out_specs=[
        pl.BlockSpec((_BM, _DH), lambda i: (i, 0)),
        pl.BlockSpec((_BM, _DH), lambda i: (i, 0)),
    ],
    out_shape=[
        jax.ShapeDtypeStruct((_N, _DH), jnp.float32),
        jax.ShapeDtypeStruct((_N, _DH), jnp.float32),
    ],
)

_mid = pl.pallas_call(
    _mid_body,
    grid=(_GRID,),
    in_specs=[
        pl.BlockSpec((_NCORES, _BM, _DH), lambda i: (0, i, 0)),
        pl.BlockSpec((_NCORES, _BM, _DH), lambda i: (0, i, 0)),
        pl.BlockSpec((_BM, _DH), lambda i: (i, 0)),
        pl.BlockSpec((_DH, _DH), lambda i: (0, 0)),
        pl.BlockSpec((_DH, _DH), lambda i: (0, 0)),
        pl.BlockSpec((1, _DH), lambda i: (0, 0)),
    ],
    out_specs=[
        pl.BlockSpec((_BM, _DH), lambda i: (i, 0)),
        pl.BlockSpec((_BM, _DH), lambda i: (i, 0)),
    ],
    out_shape=[
        jax.ShapeDtypeStruct((_N, _DH), jnp.float32),
        jax.ShapeDtypeStruct((_N, _DH), jnp.float32),
    ],
)

_outk = pl.pallas_call(
    _out_body,
    grid=(_GRID,),
    in_specs=[
        pl.BlockSpec((_NCORES, _BM, _DH), lambda i: (0, i, 0)),
        pl.BlockSpec((_NCORES, _BM, _DH), lambda i: (0, i, 0)),
        pl.BlockSpec((_BM, _DH), lambda i: (i, 0)),
    ],
    out_specs=pl.BlockSpec((_BM, _DH), lambda i: (i, 0)),
    out_shape=jax.ShapeDtypeStruct((_N, _DH), jnp.float32),
)


_mesh = plsc.VectorSubcoreMesh(core_axis_name="c", subcore_axis_name="s")


_NBUF = 8
_D = 4
_NGRP = -(-_NCHUNK // _NBUF)


def _make_seg_kernel(with_deg: bool):
    acc_t = jax.ShapeDtypeStruct((_NCORES, _NPAD, _DH), jnp.float32)
    out_type = [acc_t, acc_t] if with_deg else acc_t
    scratch = [
        pltpu.VMEM((_NCHUNK, _CHUNK), jnp.int32),
        pltpu.VMEM((_NCHUNK, _CHUNK), jnp.int32),
        pltpu.VMEM((_NBUF, _CHUNK, _DH), jnp.float32),
        pltpu.VMEM_SHARED((_NPAD, _DH), jnp.float32),
        pltpu.SemaphoreType.DMA((_NBUF,)),
        pltpu.SemaphoreType.DMA((_NBUF,)),
    ]
    if with_deg:
        scratch.insert(3, pltpu.VMEM((_CHUNK, _DH), jnp.float32))
        scratch.insert(5, pltpu.VMEM_SHARED((_NPAD, _DH), jnp.float32))
        scratch.append(pltpu.SemaphoreType.DMA((_NBUF,)))

    def body(vals_hbm, src_hbm, dst_hbm, zeros_hbm, *rest):
        if with_deg:
            (ones_hbm, out_hbm, deg_hbm, src_v, dst_v, rows_v, ones_v,
             acc_sh, deg_sh, gsem, ssem, osem) = rest
        else:
            (out_hbm, src_v, dst_v, rows_v, acc_sh, gsem, ssem) = rest
        cid = lax.axis_index("c")
        sid = lax.axis_index("s")
        wid = cid * _NSUB + sid
        pltpu.sync_copy(src_hbm.at[wid], src_v)
        pltpu.sync_copy(dst_hbm.at[wid], dst_v)
        if with_deg:
            pltpu.sync_copy(ones_hbm, ones_v)

        def fire_gather(j, b):
            pltpu.async_copy(vals_hbm.at[src_v.at[j]], rows_v.at[b],
                             gsem.at[b])

        def wait_gather(j, b):
            pltpu.make_async_copy(vals_hbm.at[src_v.at[j]], rows_v.at[b],
                                  gsem.at[b]).wait()

        def fire_scatter(j, b):
            pltpu.async_copy(rows_v.at[b], acc_sh.at[dst_v.at[j]],
                             ssem.at[b], add=True)
            if with_deg:
                pltpu.async_copy(ones_v, deg_sh.at[dst_v.at[j]],
                                 osem.at[b], add=True)

        def wait_scatter(j, b):
            pltpu.make_async_copy(rows_v.at[b], acc_sh.at[dst_v.at[j]],
                                  ssem.at[b]).wait()
            if with_deg:
                pltpu.make_async_copy(ones_v, deg_sh.at[dst_v.at[j]],
                                      osem.at[b]).wait()

        for b in range(_D):
            fire_gather(b, b)
        r0 = sid * _RPT
        pltpu.sync_copy(zeros_hbm.at[pl.ds(r0, _RPT)],
                        acc_sh.at[pl.ds(r0, _RPT)])
        if with_deg:
            pltpu.sync_copy(zeros_hbm.at[pl.ds(r0, _RPT)],
                            deg_sh.at[pl.ds(r0, _RPT)])
        plsc.subcore_barrier()

        def group(gi, carry):
            for b in range(_NBUF):
                j = gi * _NBUF + b

                @pl.when(j < _NCHUNK)
                def _turn():
                    @pl.when(j >= _D)
                    def _():
                        wait_scatter(j - _D, b ^ _D)

                    @pl.when(j + _D < _NCHUNK)
                    def _():
                        fire_gather(j + _D, b ^ _D)

                    wait_gather(j, b)
                    fire_scatter(j, b)
            return carry

        lax.fori_loop(0, _NGRP, group, 0)
        for j in range(_NCHUNK - _D, _NCHUNK):
            wait_scatter(j, j % _NBUF)
        plsc.subcore_barrier()
        pltpu.sync_copy(acc_sh.at[pl.ds(r0, _RPT)],
                        out_hbm.at[cid, pl.ds(r0, _RPT)])
        if with_deg:
            pltpu.sync_copy(deg_sh.at[pl.ds(r0, _RPT)],
                            deg_hbm.at[cid, pl.ds(r0, _RPT)])

    return functools.partial(
        pl.kernel, mesh=_mesh, out_type=out_type, scratch_types=scratch,
        compiler_params=pltpu.CompilerParams(use_tc_tiling_on_sc=False),
    )(body)


_seg_deg = _make_seg_kernel(with_deg=True)
_seg = _make_seg_kernel(with_deg=False)


def kernel(x, edge_index, W1_l, b1, W1_r, W2_l, b2, W2_r):
    f32 = jnp.float32
    src = edge_index[0].reshape(_NW, _EPT)
    dst = edge_index[1].reshape(_NW, _EPT)
    pad = _EPT_PAD - _EPT
    src = jnp.pad(src, ((0, 0), (0, pad))).reshape(_NW, _NCHUNK, _CHUNK)
    dst = jnp.pad(dst, ((0, 0), (0, pad)), constant_values=_N)
    dst = dst.reshape(_NW, _NCHUNK, _CHUNK)

    zeros = jnp.zeros((_NPAD, _DH), f32)
    ones = jnp.ones((_CHUNK, _DH), f32)

    y1, r1 = _proj1(x, W1_l, W1_r, b1.reshape(1, _DH))
    s1, deg = _seg_deg(y1, src, dst, zeros, ones)

    w2l = jnp.pad(W2_l, ((0, 0), (0, _DH - _NCLS)))
    w2r = jnp.pad(W2_r, ((0, 0), (0, _DH - _NCLS)))
    b2p = jnp.pad(b2, (0, _DH - _NCLS)).reshape(1, _DH)
    y2, r2 = _mid(s1, deg, r1, w2l, w2r, b2p)

    s2 = _seg(y2, src, dst, zeros)
    out = _outk(s2, deg, r2)
    return out[:, :_NCLS]
